# baseline (speedup 1.0000x reference)
import functools
import math

import jax
import jax.numpy as jnp
from jax import lax
from jax.experimental import pallas as pl
from jax.experimental.pallas import tpu as pltpu
from jax.experimental.pallas import tpu_sc as plsc

VOCAB = 100000
D_MODEL = 128
BATCH = 4096
HIST = 50
HIST_PAD = 56

NC = 2
NS = 16
NW = NC * NS
B_PER_W = BATCH // NW
NBUF = 8
SCALE = math.sqrt(D_MODEL)

_mesh = plsc.VectorSubcoreMesh(core_axis_name="c", subcore_axis_name="s")


@functools.partial(
    pl.kernel,
    mesh=_mesh,
    out_type=jax.ShapeDtypeStruct((BATCH, HIST, D_MODEL), jnp.float32),
    scratch_types=[
        pltpu.VMEM((B_PER_W, HIST_PAD), jnp.int32),
        pltpu.VMEM((NBUF, HIST_PAD, D_MODEL), jnp.float32),
        pltpu.SemaphoreType.DMA,
        pltpu.SemaphoreType.DMA,
    ],
)
def _emb_sc(x_hbm, w_hbm, out_hbm, idx_v, rows_v, gsem, osem):
    wid = lax.axis_index("s") * NC + lax.axis_index("c")
    b0 = wid * B_PER_W

    pltpu.sync_copy(x_hbm.at[pl.ds(b0, B_PER_W)], idx_v)

    def gather_start(j, b):
        pltpu.async_copy(w_hbm.at[idx_v.at[j]], rows_v.at[b], gsem)

    def gather_wait(j, b):
        pltpu.make_async_copy(w_hbm.at[idx_v.at[j]], rows_v.at[b], gsem).wait()

    def out_start(j, b):
        pltpu.async_copy(rows_v.at[b, pl.ds(0, HIST)], out_hbm.at[b0 + j], osem)

    def out_wait(j, b):
        pltpu.make_async_copy(
            rows_v.at[b, pl.ds(0, HIST)], out_hbm.at[b0 + j], osem
        ).wait()

    def scale_buf(b):
        rows = rows_v.at[b]

        def body(j, _):
            for i in range(D_MODEL // 16):
                sl = pl.ds(16 * i, 16)
                rows[j, sl] = rows[j, sl] * SCALE
            return 0

        lax.fori_loop(0, HIST, body, 0, unroll=2)

    for b in range(NBUF):
        gather_start(b, b)

    def outer(g, _):
        for b in range(NBUF):
            j = g * NBUF + b
            gather_wait(j, b)
            scale_buf(b)
            out_start(j, b)
            nxt = j + NBUF

            @pl.when(nxt < B_PER_W)
            def _():
                out_wait(j, b)
                gather_start(nxt, b)

        return 0

    lax.fori_loop(0, B_PER_W // NBUF, outer, 0)

    for b in range(NBUF):
        out_wait(B_PER_W - NBUF + b, b)


def kernel(x, weight):
    xp = jnp.pad(x, ((0, 0), (0, HIST_PAD - HIST)))
    return _emb_sc(xp, weight)


# speedup vs baseline: 1.0078x; 1.0078x over previous
import functools
import math

import jax
import jax.numpy as jnp
from jax import lax
from jax.experimental import pallas as pl
from jax.experimental.pallas import tpu as pltpu
from jax.experimental.pallas import tpu_sc as plsc

VOCAB = 100000
D_MODEL = 128
BATCH = 4096
HIST = 50
HIST_PAD = 56
PAIR = 2 * HIST_PAD

NC = 2
NS = 16
NW = NC * NS
B_PER_W = BATCH // NW
NPAIR = B_PER_W // 2
GRP = 4
NGRP = B_PER_W // GRP
NRB = 4
NCB = 2
SCALE = math.sqrt(D_MODEL)

_mesh = plsc.VectorSubcoreMesh(core_axis_name="c", subcore_axis_name="s")


@functools.partial(
    pl.kernel,
    mesh=_mesh,
    out_type=jax.ShapeDtypeStruct((BATCH, HIST, D_MODEL), jnp.float32),
    scratch_types=[
        pltpu.VMEM((NPAIR, PAIR), jnp.int32),
        pltpu.VMEM((NRB, PAIR, D_MODEL), jnp.float32),
        pltpu.VMEM((NCB, GRP, HIST, D_MODEL), jnp.float32),
        pltpu.SemaphoreType.DMA,
        pltpu.SemaphoreType.DMA,
    ],
)
def _emb_sc(x_hbm, w_hbm, out_hbm, idx_v, rows_v, comp_v, gsem, osem):
    wid = lax.axis_index("s") * NC + lax.axis_index("c")
    b0 = wid * B_PER_W

    pltpu.sync_copy(x_hbm.at[pl.ds(wid * NPAIR, NPAIR)], idx_v)

    def gather_start(p, rb):
        pltpu.async_copy(w_hbm.at[idx_v.at[p]], rows_v.at[rb], gsem)

    def gather_wait(p, rb):
        pltpu.make_async_copy(w_hbm.at[idx_v.at[p]], rows_v.at[rb], gsem).wait()

    def out_start(t, cb):
        pltpu.async_copy(comp_v.at[cb], out_hbm.at[pl.ds(b0 + GRP * t, GRP)], osem)

    def out_wait(t, cb):
        pltpu.make_async_copy(
            comp_v.at[cb], out_hbm.at[pl.ds(b0 + GRP * t, GRP)], osem
        ).wait()

    def scale_group(cb, rb_even, rb_odd):
        comp = comp_v.at[cb]
        for g in range(GRP):
            rows = rows_v.at[rb_even if g < 2 else rb_odd]
            base = HIST_PAD * (g % 2)

            def body(k, _):
                for i in range(D_MODEL // 16):
                    sl = pl.ds(16 * i, 16)
                    comp[g, k, sl] = rows[base + k, sl] * SCALE
                return 0

            lax.fori_loop(0, HIST, body, 0, unroll=2)

    for rb in range(NRB):
        gather_start(rb, rb)

    def outer(tt, _):
        for u in range(NCB):
            t = tt * NCB + u
            p0 = 2 * t
            rb0 = (2 * u) % NRB
            rb1 = (2 * u + 1) % NRB
            cb = u
            gather_wait(p0, rb0)
            gather_wait(p0 + 1, rb1)

            @pl.when(t >= NCB)
            def _():
                out_wait(t - NCB, cb)

            scale_group(cb, rb0, rb1)
            out_start(t, cb)

            @pl.when(p0 + NRB + 1 < NPAIR + 1)
            def _():
                gather_start(p0 + NRB, rb0)
                gather_start(p0 + NRB + 1, rb1)

        return 0

    lax.fori_loop(0, NGRP // NCB, outer, 0)

    for u in range(NCB):
        out_wait(NGRP - NCB + u, u)


def kernel(x, weight):
    xp = jnp.pad(x, ((0, 0), (0, HIST_PAD - HIST))).reshape(BATCH // 2, PAIR)
    return _emb_sc(xp, weight)
